# Initial kernel scaffold; baseline (speedup 1.0000x reference)
import jax
import jax.numpy as jnp
from jax.experimental import pallas as pl


def kernel(x, table):
    raise NotImplementedError("write your pallas kernel here")


# speedup vs baseline: 9.1675x; 9.1675x over previous
import functools

import jax
import jax.numpy as jnp
from jax import lax
from jax.experimental import pallas as pl
from jax.experimental.pallas import tpu as pltpu
from jax.experimental.pallas import tpu_sc as plsc

NW = 32
CHUNK = 128
NBUF = 4


def _gather_kernel(n_rows, d, nch, table_hbm, idx_hbm, out_hbm, idx_v, bufs,
                   gsem, ssem):
    wid = lax.axis_index("s") * 2 + lax.axis_index("c")
    base = wid * (nch * CHUNK)
    pltpu.sync_copy(idx_hbm.at[wid], idx_v)

    def gather(g, b):
        return pltpu.make_async_copy(table_hbm.at[idx_v.at[g]], bufs.at[b],
                                     gsem.at[b])

    def scatter(g, b):
        return pltpu.make_async_copy(
            bufs.at[b], out_hbm.at[pl.ds(base + g * CHUNK, CHUNK)],
            ssem.at[b])

    for b in range(NBUF):
        gather(b, b).start()

    ngrp = nch // NBUF

    def group(o, carry):
        for b in range(NBUF):
            g = o * NBUF + b
            gather(g, b).wait()
            scatter(g, b).start()
        for b in range(NBUF):
            g = o * NBUF + b
            scatter(g, b).wait()
            gather(g + NBUF, b).start()
        return carry

    lax.fori_loop(0, ngrp - 1, group, 0, unroll=False)

    for b in range(NBUF):
        g = (ngrp - 1) * NBUF + b
        gather(g, b).wait()
        scatter(g, b).start()
    for b in range(NBUF):
        g = (ngrp - 1) * NBUF + b
        scatter(g, b).wait()


def kernel(x, table):
    bsz, seq = x.shape
    v, d = table.shape
    total = bsz * seq
    per_w = total // NW
    nch = per_w // CHUNK
    idx = x.reshape(NW, nch, CHUNK).astype(jnp.int32)

    mesh = plsc.VectorSubcoreMesh(core_axis_name="c", subcore_axis_name="s")
    k = functools.partial(
        pl.kernel,
        mesh=mesh,
        out_type=jax.ShapeDtypeStruct((total, d), jnp.float32),
        scratch_types=[
            pltpu.VMEM((nch, CHUNK), jnp.int32),
            pltpu.VMEM((NBUF, CHUNK, d), jnp.float32),
            pltpu.SemaphoreType.DMA((NBUF,)),
            pltpu.SemaphoreType.DMA((NBUF,)),
        ],
    )(functools.partial(_gather_kernel, total, d, nch))
    out = k(table, idx)
    return out.reshape(bsz, seq, d)
